# baseline (speedup 1.0000x reference)
import functools

import jax
import jax.numpy as jnp
from jax import lax
from jax.experimental import pallas as pl
from jax.experimental.pallas import tpu as pltpu
from jax.experimental.pallas import tpu_sc as plsc


def _sc_gather_pairs(table2, idx, chunk=512):
    n_rows = idx.shape[0]
    d2 = table2.shape[1]
    info = plsc.get_sparse_core_info()
    nl = info.num_lanes
    nw = info.num_cores * info.num_subcores
    per_w = n_rows // nw
    n_chunks = per_w // chunk
    assert per_w % chunk == 0 and n_rows % nw == 0 and chunk % nl == 0

    mesh = plsc.VectorSubcoreMesh(core_axis_name="c", subcore_axis_name="s")

    @functools.partial(
        pl.kernel,
        out_type=jax.ShapeDtypeStruct((n_rows, d2), jnp.float32),
        mesh=mesh,
        scratch_types=[
            pltpu.VMEM((chunk,), jnp.int32),
            pltpu.VMEM((chunk,), jnp.int32),
            pltpu.VMEM((chunk, d2), jnp.float32),
            pltpu.SemaphoreType.DMA,
        ],
    )
    def gather_kernel(table_hbm, idx_hbm, out_hbm, idx_v, idxh_v, rows_v, sem):
        wid = lax.axis_index("s") * info.num_cores + lax.axis_index("c")
        base = wid * per_w

        @pl.loop(0, n_chunks)
        def _(c):
            off = base + c * chunk
            pltpu.sync_copy(idx_hbm.at[pl.ds(off, chunk)], idx_v)
            for u in range(chunk // nl):
                sl = pl.ds(u * nl, nl)
                idxh_v[sl] = jax.lax.shift_right_logical(idx_v[sl], 1)
            pltpu.async_copy(table_hbm.at[idxh_v], rows_v, sem).wait()
            pltpu.sync_copy(rows_v, out_hbm.at[pl.ds(off, chunk)])

    return gather_kernel(table2, idx)


def _tc_select_linear(g, idx, w, bias, batch, fields, d, blk=4096):
    n = g.shape[0]

    def body(g_ref, i_ref, w_ref, b_ref, o_ref):
        x128 = g_ref[...]
        h = lax.broadcast_in_dim(i_ref[...] & 1, (blk, d), (0,))
        x = jnp.where(h == 1, x128[:, d:], x128[:, :d])
        o_ref[...] = lax.dot_general(
            x, w_ref[...],
            (((1,), (1,)), ((), ())),
            preferred_element_type=jnp.float32,
        ) + b_ref[...]

    out = pl.pallas_call(
        body,
        grid=(n // blk,),
        in_specs=[
            pl.BlockSpec((blk, 2 * d), lambda i: (i, 0)),
            pl.BlockSpec((blk,), lambda i: (i,)),
            pl.BlockSpec((d, d), lambda i: (0, 0)),
            pl.BlockSpec((1, d), lambda i: (0, 0)),
        ],
        out_specs=pl.BlockSpec((blk, d), lambda i: (i, 0)),
        out_shape=jax.ShapeDtypeStruct((n, d), jnp.float32),
    )(g, idx, w, bias)
    return out.reshape(batch, fields, d)


def kernel(group_id, table, W, b):
    batch, fields = group_id.shape
    n_vocab, d = table.shape
    idx = group_id.reshape(-1).astype(jnp.int32)
    table2 = table.reshape(n_vocab // 2, 2 * d)
    g = _sc_gather_pairs(table2, idx)
    return _tc_select_linear(g, idx, W, b.reshape(1, d), batch, fields, d)


# speedup vs baseline: 1.1035x; 1.1035x over previous
import functools

import jax
import jax.numpy as jnp
from jax import lax
from jax.experimental import pallas as pl
from jax.experimental.pallas import tpu as pltpu
from jax.experimental.pallas import tpu_sc as plsc


def _sc_gather_pairs(table2, idx, chunk=512):
    n_rows = idx.shape[0]
    d2 = table2.shape[1]
    info = plsc.get_sparse_core_info()
    nl = info.num_lanes
    nw = info.num_cores * info.num_subcores
    per_w = n_rows // nw
    n_chunks = per_w // chunk
    assert per_w % chunk == 0 and n_rows % nw == 0 and chunk % nl == 0

    mesh = plsc.VectorSubcoreMesh(core_axis_name="c", subcore_axis_name="s")

    @functools.partial(
        pl.kernel,
        out_type=jax.ShapeDtypeStruct((n_rows, d2), jnp.float32),
        mesh=mesh,
        scratch_types=[
            pltpu.VMEM((chunk,), jnp.int32),
            pltpu.VMEM((chunk,), jnp.int32),
            pltpu.VMEM((chunk, d2), jnp.float32),
            pltpu.SemaphoreType.DMA,
        ],
    )
    def gather_kernel(table_hbm, idx_hbm, out_hbm, idx_v, idxh_v, rows_v, sem):
        wid = lax.axis_index("s") * info.num_cores + lax.axis_index("c")
        base = wid * per_w

        @pl.loop(0, n_chunks)
        def _(c):
            off = base + c * chunk
            pltpu.sync_copy(idx_hbm.at[pl.ds(off, chunk)], idx_v)
            for u in range(chunk // nl):
                sl = pl.ds(u * nl, nl)
                idxh_v[sl] = jax.lax.shift_right_logical(idx_v[sl], 1)
            pltpu.async_copy(table_hbm.at[idxh_v], rows_v, sem).wait()
            pltpu.sync_copy(rows_v, out_hbm.at[pl.ds(off, chunk)])

    return gather_kernel(table2, idx)


def _tc_select_linear(g, idx, w, bias, batch, fields, d, bb=512):
    blk = bb * fields

    def body(g_ref, i_ref, w_ref, b_ref, o_ref):
        x128 = g_ref[...]
        h = lax.broadcast_in_dim(i_ref[...] & 1, (blk, d), (0,))
        x = jnp.where(h == 1, x128[:, d:], x128[:, :d])
        y = lax.dot_general(
            x, w_ref[...],
            (((1,), (1,)), ((), ())),
            preferred_element_type=jnp.float32,
        ) + b_ref[...]
        o_ref[...] = y.reshape(bb, fields, d)

    return pl.pallas_call(
        body,
        grid=(batch // bb,),
        in_specs=[
            pl.BlockSpec((blk, 2 * d), lambda i: (i, 0)),
            pl.BlockSpec((blk,), lambda i: (i,)),
            pl.BlockSpec((d, d), lambda i: (0, 0)),
            pl.BlockSpec((1, d), lambda i: (0, 0)),
        ],
        out_specs=pl.BlockSpec((bb, fields, d), lambda i: (i, 0, 0)),
        out_shape=jax.ShapeDtypeStruct((batch, fields, d), jnp.float32),
    )(g, idx, w, bias)


def kernel(group_id, table, W, b):
    batch, fields = group_id.shape
    n_vocab, d = table.shape
    idx = group_id.reshape(-1).astype(jnp.int32)
    table2 = table.reshape(n_vocab // 2, 2 * d)
    g = _sc_gather_pairs(table2, idx)
    return _tc_select_linear(g, idx, W, b.reshape(1, d), batch, fields, d)
